# baseline (speedup 1.0000x reference)
import jax
import jax.numpy as jnp
from jax import lax
from jax.experimental import pallas as pl
from jax.experimental.pallas import tpu as pltpu

N = 16384
F = 512
H = 128
S = 8
BLK = 1024
NBLK = N // BLK


def _body(b2_ref, b3_ref, fnext_ref, last_ref, W0_ref, b0_ref, W1_ref,
          b1_ref, x_hbm, out_hbm, xbuf_ref, gate_ref, rsem, wsem):
    read_copies = []
    for k in range(NBLK):
        c = pltpu.make_async_copy(
            x_hbm.at[pl.ds(k * BLK, BLK), :],
            xbuf_ref.at[pl.ds(k * BLK, BLK), :],
            rsem.at[k])
        c.start()
        read_copies.append(c)

    b2 = b2_ref[...]
    cnt = jnp.concatenate(
        [jnp.sum((b2 == s).astype(jnp.float32))[None] for s in range(S)])
    inv_cnt = 1.0 / jnp.maximum(cnt, 1.0)[:, None]

    W0 = W0_ref[...]
    b0 = b0_ref[...]
    W1 = W1_ref[...]
    b1 = b1_ref[...]

    def onehot_static(k):
        ids = b3_ref[k, 0, :]
        return (ids[:, None] == lax.broadcasted_iota(jnp.int32, (BLK, S), 1)
                ).astype(jnp.float32)

    def write_block(w, gate):
        ids = b3_ref[pl.ds(w, 1), 0, :].reshape(BLK)
        oh = (ids[:, None] == lax.broadcasted_iota(jnp.int32, (BLK, S), 1)
              ).astype(jnp.float32)
        y = lax.dot_general(oh, gate, (((1,), (0,)), ((), ())),
                            preferred_element_type=jnp.float32)
        xbuf_ref[pl.ds(w * BLK, BLK), :] *= y
        pltpu.make_async_copy(
            xbuf_ref.at[pl.ds(w * BLK, BLK), :],
            out_hbm.at[pl.ds(w * BLK, BLK), :],
            wsem.at[w]).start()

    acc = jnp.zeros((S, F), jnp.float32)
    wptr = jnp.int32(0)
    frontier = jnp.int32(0)
    for r in range(NBLK):
        read_copies[r].wait()
        xi = xbuf_ref[pl.ds(r * BLK, BLK), :]
        acc = acc + lax.dot_general(onehot_static(r), xi,
                                    (((0,), (0,)), ((), ())),
                                    preferred_element_type=jnp.float32)
        new_frontier = fnext_ref[r]

        @pl.when(new_frontier > frontier)
        def _():
            mean = acc * inv_cnt
            h = jnp.maximum(
                lax.dot_general(mean, W0, (((1,), (0,)), ((), ())),
                                preferred_element_type=jnp.float32) + b0,
                0.0)
            z = lax.dot_general(h, W1, (((1,), (0,)), ((), ())),
                                preferred_element_type=jnp.float32) + b1
            gate_ref[...] = 1.0 / (1.0 + jnp.exp(-z))

        gate = gate_ref[...]

        def cond(w):
            return (w < NBLK) & (last_ref[jnp.minimum(w, NBLK - 1)]
                                 < new_frontier)

        def body(w):
            write_block(w, gate)
            return w + 1

        wptr = lax.while_loop(cond, body, wptr)
        frontier = new_frontier

    for k in range(NBLK):
        pltpu.make_async_copy(
            xbuf_ref.at[pl.ds(k * BLK, BLK), :],
            out_hbm.at[pl.ds(k * BLK, BLK), :],
            wsem.at[k]).wait()


def kernel(x, batch, W0, b0, W1, b1):
    batch32 = batch.astype(jnp.int32)
    b3 = batch32.reshape(NBLK, 1, BLK)
    b2 = batch32.reshape(128, 128)
    fnext = jnp.concatenate(
        [batch32[BLK::BLK], jnp.full((1,), S, jnp.int32)])
    last = batch32[BLK - 1::BLK]

    out = pl.pallas_call(
        _body,
        in_specs=[
            pl.BlockSpec(memory_space=pltpu.MemorySpace.VMEM),
            pl.BlockSpec(memory_space=pltpu.MemorySpace.VMEM),
            pl.BlockSpec(memory_space=pltpu.MemorySpace.SMEM),
            pl.BlockSpec(memory_space=pltpu.MemorySpace.SMEM),
            pl.BlockSpec(memory_space=pltpu.MemorySpace.VMEM),
            pl.BlockSpec(memory_space=pltpu.MemorySpace.VMEM),
            pl.BlockSpec(memory_space=pltpu.MemorySpace.VMEM),
            pl.BlockSpec(memory_space=pltpu.MemorySpace.VMEM),
            pl.BlockSpec(memory_space=pltpu.MemorySpace.HBM),
        ],
        out_specs=pl.BlockSpec(memory_space=pltpu.MemorySpace.HBM),
        out_shape=jax.ShapeDtypeStruct((N, F), jnp.float32),
        scratch_shapes=[
            pltpu.VMEM((N, F), jnp.float32),
            pltpu.VMEM((S, F), jnp.float32),
            pltpu.SemaphoreType.DMA((NBLK,)),
            pltpu.SemaphoreType.DMA((NBLK,)),
        ],
    )(b2, b3, fnext, last, W0, b0.reshape(1, H), W1, b1.reshape(1, F), x)

    return out


# speedup vs baseline: 1.0383x; 1.0383x over previous
import jax
import jax.numpy as jnp
from jax import lax
from jax.experimental import pallas as pl
from jax.experimental.pallas import tpu as pltpu

N = 16384
F = 512
H = 128
S = 8
BLK = 1024
NBLK = N // BLK


def _body(b2_ref, b3_ref, W0_ref, b0_ref, W1_ref, b1_ref, x_hbm, out_hbm,
          xbuf_ref, rsem, wsem):
    read_copies = []
    for k in range(NBLK):
        c = pltpu.make_async_copy(
            x_hbm.at[pl.ds(k * BLK, BLK), :],
            xbuf_ref.at[pl.ds(k * BLK, BLK), :],
            rsem.at[k])
        c.start()
        read_copies.append(c)

    b2 = b2_ref[...]
    cnt = jnp.concatenate(
        [jnp.sum((b2 == s).astype(jnp.float32))[None] for s in range(S)])
    inv_cnt = 1.0 / jnp.maximum(cnt, 1.0)[:, None]
    W0 = W0_ref[...]
    b0 = b0_ref[...]
    W1 = W1_ref[...]
    b1 = b1_ref[...]

    def onehot(k):
        ids = b3_ref[k, 0, :]
        return (ids[:, None] == lax.broadcasted_iota(jnp.int32, (BLK, S), 1)
                ).astype(jnp.float32)

    acc = jnp.zeros((S, F), jnp.float32)
    for k in range(NBLK):
        read_copies[k].wait()
        xi = xbuf_ref[pl.ds(k * BLK, BLK), :]
        acc = acc + lax.dot_general(onehot(k), xi, (((0,), (0,)), ((), ())),
                                    preferred_element_type=jnp.float32)

    mean = acc * inv_cnt
    h = jnp.maximum(
        lax.dot_general(mean, W0, (((1,), (0,)), ((), ())),
                        preferred_element_type=jnp.float32) + b0, 0.0)
    z = lax.dot_general(h, W1, (((1,), (0,)), ((), ())),
                        preferred_element_type=jnp.float32) + b1
    gate = 1.0 / (1.0 + jnp.exp(-z))

    write_copies = []
    for k in range(NBLK):
        y = lax.dot_general(onehot(k), gate, (((1,), (0,)), ((), ())),
                            preferred_element_type=jnp.float32)
        xbuf_ref[pl.ds(k * BLK, BLK), :] *= y
        c = pltpu.make_async_copy(
            xbuf_ref.at[pl.ds(k * BLK, BLK), :],
            out_hbm.at[pl.ds(k * BLK, BLK), :],
            wsem.at[k])
        c.start()
        write_copies.append(c)

    for c in write_copies:
        c.wait()


def kernel(x, batch, W0, b0, W1, b1):
    batch32 = batch.astype(jnp.int32)
    b3 = batch32.reshape(NBLK, 1, BLK)
    b2 = batch32.reshape(128, 128)

    out = pl.pallas_call(
        _body,
        in_specs=[
            pl.BlockSpec(memory_space=pltpu.MemorySpace.VMEM),
            pl.BlockSpec(memory_space=pltpu.MemorySpace.VMEM),
            pl.BlockSpec(memory_space=pltpu.MemorySpace.VMEM),
            pl.BlockSpec(memory_space=pltpu.MemorySpace.VMEM),
            pl.BlockSpec(memory_space=pltpu.MemorySpace.VMEM),
            pl.BlockSpec(memory_space=pltpu.MemorySpace.VMEM),
            pl.BlockSpec(memory_space=pltpu.MemorySpace.HBM),
        ],
        out_specs=pl.BlockSpec(memory_space=pltpu.MemorySpace.HBM),
        out_shape=jax.ShapeDtypeStruct((N, F), jnp.float32),
        scratch_shapes=[
            pltpu.VMEM((N, F), jnp.float32),
            pltpu.SemaphoreType.DMA((NBLK,)),
            pltpu.SemaphoreType.DMA((NBLK,)),
        ],
    )(b2, b3, W0, b0.reshape(1, H), W1, b1.reshape(1, F), x)

    return out


# speedup vs baseline: 1.0389x; 1.0005x over previous
import jax
import jax.numpy as jnp
from jax import lax
from jax.experimental import pallas as pl
from jax.experimental.pallas import tpu as pltpu

N = 16384
F = 512
H = 128
S = 8
BLK = 512
NBLK = N // BLK


def _body(b2_ref, b3_ref, W0_ref, b0_ref, W1_ref, b1_ref, x_hbm, out_hbm,
          xbuf_ref, rsem, wsem):
    read_copies = []
    for k in range(NBLK):
        c = pltpu.make_async_copy(
            x_hbm.at[pl.ds(k * BLK, BLK), :],
            xbuf_ref.at[pl.ds(k * BLK, BLK), :],
            rsem.at[k])
        c.start()
        read_copies.append(c)

    b2 = b2_ref[...]
    cnt = jnp.concatenate(
        [jnp.sum((b2 == s).astype(jnp.float32))[None] for s in range(S)])
    inv_cnt = 1.0 / jnp.maximum(cnt, 1.0)[:, None]
    W0 = W0_ref[...]
    b0 = b0_ref[...]
    W1 = W1_ref[...]
    b1 = b1_ref[...]

    def onehot(k):
        ids = b3_ref[k, 0, :]
        return (ids[:, None] == lax.broadcasted_iota(jnp.int32, (BLK, S), 1)
                ).astype(jnp.float32)

    acc = jnp.zeros((S, F), jnp.float32)
    for k in range(NBLK):
        read_copies[k].wait()
        xi = xbuf_ref[pl.ds(k * BLK, BLK), :]
        acc = acc + lax.dot_general(onehot(k), xi, (((0,), (0,)), ((), ())),
                                    preferred_element_type=jnp.float32)

    mean = acc * inv_cnt
    h = jnp.maximum(
        lax.dot_general(mean, W0, (((1,), (0,)), ((), ())),
                        preferred_element_type=jnp.float32) + b0, 0.0)
    z = lax.dot_general(h, W1, (((1,), (0,)), ((), ())),
                        preferred_element_type=jnp.float32) + b1
    gate = 1.0 / (1.0 + jnp.exp(-z))

    write_copies = []
    for k in range(NBLK):
        y = lax.dot_general(onehot(k), gate, (((1,), (0,)), ((), ())),
                            preferred_element_type=jnp.float32)
        xbuf_ref[pl.ds(k * BLK, BLK), :] *= y
        c = pltpu.make_async_copy(
            xbuf_ref.at[pl.ds(k * BLK, BLK), :],
            out_hbm.at[pl.ds(k * BLK, BLK), :],
            wsem.at[k])
        c.start()
        write_copies.append(c)

    for c in write_copies:
        c.wait()


def kernel(x, batch, W0, b0, W1, b1):
    batch32 = batch.astype(jnp.int32)
    b3 = batch32.reshape(NBLK, 1, BLK)
    b2 = batch32.reshape(128, 128)

    out = pl.pallas_call(
        _body,
        in_specs=[
            pl.BlockSpec(memory_space=pltpu.MemorySpace.VMEM),
            pl.BlockSpec(memory_space=pltpu.MemorySpace.VMEM),
            pl.BlockSpec(memory_space=pltpu.MemorySpace.VMEM),
            pl.BlockSpec(memory_space=pltpu.MemorySpace.VMEM),
            pl.BlockSpec(memory_space=pltpu.MemorySpace.VMEM),
            pl.BlockSpec(memory_space=pltpu.MemorySpace.VMEM),
            pl.BlockSpec(memory_space=pltpu.MemorySpace.HBM),
        ],
        out_specs=pl.BlockSpec(memory_space=pltpu.MemorySpace.HBM),
        out_shape=jax.ShapeDtypeStruct((N, F), jnp.float32),
        scratch_shapes=[
            pltpu.VMEM((N, F), jnp.float32),
            pltpu.SemaphoreType.DMA((NBLK,)),
            pltpu.SemaphoreType.DMA((NBLK,)),
        ],
    )(b2, b3, W0, b0.reshape(1, H), W1, b1.reshape(1, F), x)

    return out
